# baseline (speedup 1.0000x reference)
import dataclasses

import jax
import jax.numpy as jnp
from jax import lax
from jax.experimental import pallas as pl
from jax.experimental.pallas import tpu as pltpu
from jax.experimental.pallas import tpu_sc as plsc

N = 10000
D = 128
E = 320000

NUM_CORES = 2
NUM_SUBCORES = 16
CHUNK = 128
NCH = 80
EPC_PAD = NUM_SUBCORES * NCH * CHUNK
PAD = EPC_PAD - E // NUM_CORES
OWN_ROWS = 1000
ZROWS = 40


def _sc_aggregate(x, pk):
    mesh = plsc.VectorSubcoreMesh(core_axis_name="c", subcore_axis_name="s")
    cp = pltpu.CompilerParams()
    if "needs_layout_passes" in pltpu.CompilerParams.__dataclass_fields__:
        cp = dataclasses.replace(cp, needs_layout_passes=False)

    @pl.kernel(
        out_type=jax.ShapeDtypeStruct((NUM_CORES, N, D), jnp.float32),
        mesh=mesh,
        compiler_params=cp,
        scratch_types=[
            pltpu.VMEM_SHARED((N, D), jnp.float32),
            pltpu.VMEM((CHUNK, D), jnp.float32),
            pltpu.VMEM((CHUNK, D), jnp.float32),
            pltpu.VMEM((3, CHUNK), jnp.int32),
            pltpu.VMEM((3, CHUNK), jnp.int32),
            pltpu.SemaphoreType.DMA,
            pltpu.SemaphoreType.DMA,
        ],
    )
    def agg_kernel(x_hbm, pk_hbm, out_hbm,
                   acc, rows0, rows1, pk0, pk1, sem0, sem1):
        cid = lax.axis_index("c")
        sid = lax.axis_index("s")
        rows_v = rows0

        @pl.when(sid < N // OWN_ROWS)
        def _():
            zero16 = jnp.zeros((16,), jnp.float32)
            for r in range(ZROWS):
                for j in range(D // 16):
                    rows_v[r, pl.ds(j * 16, 16)] = zero16
            base_row = pl.multiple_of(sid * OWN_ROWS, 8)

            @pl.loop(0, OWN_ROWS, step=ZROWS)
            def _(t):
                pltpu.sync_copy(rows_v.at[pl.ds(0, ZROWS)],
                                acc.at[pl.ds(base_row + t, ZROWS)])

        plsc.subcore_barrier()

        ebase = pl.multiple_of(cid * EPC_PAD + sid * (NCH * CHUNK), 128)

        def load_pk(c, pkb):
            bb = pl.multiple_of(ebase + c * CHUNK, 128)
            pltpu.sync_copy(pk_hbm.at[:, pl.ds(bb, CHUNK)], pkb)

        def scale(rowsb, pkb):
            @pl.loop(0, CHUNK, step=16)
            def _(g):
                wg = plsc.bitcast(pkb[2, pl.ds(g, 16)], jnp.float32)
                for k in range(16):
                    wi = wg[k]
                    for j in range(D // 16):
                        sl = pl.ds(j * 16, 16)
                        rowsb[g + k, sl] = rowsb[g + k, sl] * wi

        load_pk(0, pk0)
        pltpu.make_async_copy(x_hbm.at[pk0.at[0]], rows0, sem0).start()

        @pl.loop(0, NCH, step=2)
        def _(k):
            bufs = ((rows0, pk0, sem0), (rows1, pk1, sem1))
            for b in range(2):
                rowsb, pkb, semb = bufs[b]
                rowsn, pkn, semn = bufs[1 - b]
                c = k + b

                @pl.when(c + 1 < NCH)
                def _():
                    load_pk(c + 1, pkn)
                    pltpu.make_async_copy(
                        x_hbm.at[pkn.at[0]], rowsn, semn).start()

                pltpu.make_async_copy(x_hbm.at[pkb.at[0]], rowsb, semb).wait()
                scale(rowsb, pkb)
                pltpu.sync_copy(rowsb, acc.at[pkb.at[1]], add=True)

        plsc.subcore_barrier()

        @pl.when(sid < N // OWN_ROWS)
        def _():
            base_row = pl.multiple_of(sid * OWN_ROWS, 8)
            pltpu.sync_copy(acc.at[pl.ds(base_row, OWN_ROWS)],
                            out_hbm.at[cid].at[pl.ds(base_row, OWN_ROWS)])

    return agg_kernel(x, pk)


def _tc_mlp(x, partials, W1x, W1a, b1, W2, b2):
    BLK = 2000

    def body(x_ref, p0_ref, p1_ref, W1x_ref, W1a_ref, b1_ref, W2_ref, b2_ref,
             o_ref):
        agg = p0_ref[0] + p1_ref[0]
        h = jnp.dot(x_ref[...], W1x_ref[...], preferred_element_type=jnp.float32)
        h += jnp.dot(agg, W1a_ref[...], preferred_element_type=jnp.float32)
        h = jnp.maximum(h + b1_ref[...], 0.0)
        o_ref[...] = (
            jnp.dot(h, W2_ref[...], preferred_element_type=jnp.float32)
            + b2_ref[...]
        )

    full = lambda i: (0, 0)
    return pl.pallas_call(
        body,
        grid=(N // BLK,),
        in_specs=[
            pl.BlockSpec((BLK, D), lambda i: (i, 0)),
            pl.BlockSpec((1, BLK, D), lambda i: (0, i, 0)),
            pl.BlockSpec((1, BLK, D), lambda i: (1, i, 0)),
            pl.BlockSpec((D, D), full),
            pl.BlockSpec((D, D), full),
            pl.BlockSpec((1, D), full),
            pl.BlockSpec((D, D), full),
            pl.BlockSpec((1, D), full),
        ],
        out_specs=pl.BlockSpec((BLK, D), lambda i: (i, 0)),
        out_shape=jax.ShapeDtypeStruct((N, D), jnp.float32),
    )(x, partials, partials, W1x, W1a, b1, W2, b2)


def kernel(x, edge_index, edge_weight, W1, b1, W2, b2):
    src = edge_index[1].astype(jnp.int32)
    dst = edge_index[0].astype(jnp.int32)
    w_i32 = jax.lax.bitcast_convert_type(edge_weight, jnp.int32)
    pk = jnp.stack([src, dst, w_i32])
    half = E // NUM_CORES
    z = jnp.zeros((3, PAD), jnp.int32)
    pk = jnp.concatenate([pk[:, :half], z, pk[:, half:], z], axis=1)
    partials = _sc_aggregate(x, pk)
    W1x = W1[:D]
    W1a = W1[D:]
    return _tc_mlp(x, partials, W1x, W1a, b1.reshape(1, D), W2,
                   b2.reshape(1, D))
